# baseline (speedup 1.0000x reference)
import jax
import jax.numpy as jnp
from jax import lax
from jax.experimental import pallas as pl
from jax.experimental.pallas import tpu as pltpu
from jax.experimental.pallas import tpu_sc as plsc


A_SZ = 16
N_SEL = 24
K = 2

_NW = 32
_NPW = 8192 // _NW
_G = 8
_NG = _NPW // _G


def _select_body(a_ref, r_ref, p_ref, o_ref):
    a = a_ref[...]
    ap = jnp.max(a, axis=1)
    qi = lax.broadcasted_iota(jnp.int32, (32, 4, 64), 1)
    argq = jnp.min(jnp.where(a == ap[:, None, :], qi, 4), axis=1)
    wi = lax.broadcasted_iota(jnp.int32, (32, 64, 64), 1)
    wj = lax.broadcasted_iota(jnp.int32, (32, 64, 64), 2)
    vi = ap[:, :, None]
    vj = ap[:, None, :]
    beats = (vj > vi) | ((vj == vi) & (wj < wi))
    arank = jnp.sum(beats.astype(jnp.int32), axis=2)
    m4 = ((arank < A_SZ)[:, None, :] & (argq[:, None, :] == qi)).astype(
        jnp.float32
    )

    r = r_ref[...].reshape(8, 32, 4, 64)
    rm = r + m4[None]
    rp = jnp.max(rm, axis=2)
    qi2 = lax.broadcasted_iota(jnp.int32, (8, 32, 4, 64), 2)
    argq2 = jnp.min(jnp.where(rm == rp[:, :, None, :], qi2, 4), axis=2)
    wio = lax.broadcasted_iota(jnp.int32, (8, 32, 64), 2)
    di = argq2 // K
    dj = argq2 % K
    flat = (K * (wio // 8) + di) * 16 + (K * (wio % 8) + dj)

    rpf = rp.reshape(256, 64)
    wi2 = lax.broadcasted_iota(jnp.int32, (256, 64, 64), 1)
    wj2 = lax.broadcasted_iota(jnp.int32, (256, 64, 64), 2)
    vi2 = rpf[:, :, None]
    vj2 = rpf[:, None, :]
    beats2 = (vj2 > vi2) | ((vj2 == vi2) & (wj2 < wi2))
    rrank = jnp.sum(beats2.astype(jnp.int32), axis=2)

    onehot = (rrank[:, None, :] == p_ref[...][None, :, :]).astype(jnp.float32)
    flatf = flat.reshape(256, 64).astype(jnp.float32)
    sel = jnp.sum(flatf[:, None, :] * onehot, axis=2)
    o_ref[...] = sel.astype(jnp.int32)


def _sc_gather_body(votes_hbm, sel_hbm, out_hbm, idxv, vbuf, obuf, sem_in, sem_out):
    c = lax.axis_index("c")
    s = lax.axis_index("s")
    wid = s * 2 + c
    base = wid * _NPW
    pltpu.sync_copy(sel_hbm.at[pl.ds(base, _NPW)], idxv)
    pltpu.async_copy(votes_hbm.at[pl.ds(base, _G)], vbuf.at[0], sem_in)

    def do_group(g, par):
        @pl.when(g + 1 < _NG)
        def _():
            pltpu.async_copy(
                votes_hbm.at[pl.ds(base + (g + 1) * _G, _G)], vbuf.at[1 - par], sem_in
            )

        pltpu.make_async_copy(
            votes_hbm.at[pl.ds(base + g * _G, _G)], vbuf.at[par], sem_in
        ).wait()

        @pl.when(g >= 2)
        def _():
            pltpu.make_async_copy(
                obuf.at[par], out_hbm.at[pl.ds(base + (g - 2) * _G, _G)], sem_out
            ).wait()

        for e in range(_G):
            n_local = g * _G + e
            ir0 = idxv[n_local, pl.ds(0, 16)]
            ir1 = idxv[n_local, pl.ds(16, 16)]
            vtab = vbuf.at[par, e]
            for h in range(16):
                hsp = jnp.full((16,), h, jnp.int32)
                obuf[par, e, h, pl.ds(0, 16)] = plsc.load_gather(vtab, [hsp, ir0])
                obuf[par, e, h, pl.ds(16, 16)] = plsc.load_gather(vtab, [hsp, ir1])
        pltpu.async_copy(obuf.at[par], out_hbm.at[pl.ds(base + g * _G, _G)], sem_out)

    def it_body(i, carry):
        do_group(2 * i, 0)
        do_group(2 * i + 1, 1)
        return carry

    lax.fori_loop(0, _NG // 2, it_body, 0)
    pltpu.make_async_copy(
        obuf.at[0], out_hbm.at[pl.ds(base + (_NG - 2) * _G, _G)], sem_out
    ).wait()
    pltpu.make_async_copy(
        obuf.at[1], out_hbm.at[pl.ds(base + (_NG - 1) * _G, _G)], sem_out
    ).wait()


def _compute_sel(a4, r4, perm2d, interpret=False):
    return pl.pallas_call(
        _select_body,
        grid=(32,),
        in_specs=[
            pl.BlockSpec((32, 4, 64), lambda t: (t // 4, 0, 0)),
            pl.BlockSpec((256, 4, 64), lambda t: (t, 0, 0)),
            pl.BlockSpec((32, 64), lambda t: (0, 0)),
        ],
        out_specs=pl.BlockSpec((256, 32), lambda t: (t, 0)),
        out_shape=jax.ShapeDtypeStruct((8192, 32), jnp.int32),
        interpret=interpret,
    )(a4, r4, perm2d)


def _gather_votes(votes_r, sel):
    mesh = plsc.VectorSubcoreMesh(core_axis_name="c", subcore_axis_name="s")
    f = pl.kernel(
        _sc_gather_body,
        out_type=jax.ShapeDtypeStruct((8192, 16, 32), jnp.float32),
        mesh=mesh,
        scratch_types=[
            pltpu.VMEM((_NPW, 32), jnp.int32),
            pltpu.VMEM((2, _G, 16, 256), jnp.float32),
            pltpu.VMEM((2, _G, 16, 32), jnp.float32),
            pltpu.SemaphoreType.DMA,
            pltpu.SemaphoreType.DMA,
        ],
        compiler_params=pltpu.CompilerParams(
            use_tc_tiling_on_sc=False, needs_layout_passes=False
        ),
    )
    return f(votes_r, sel)


@jax.jit
def kernel(x, route, votes):
    b, idim, odim, h, dx, dy = votes.shape
    a_orig = x[..., h - 1]
    a4 = (
        a_orig.reshape(b * odim, dx // K, K, dy // K, K)
        .transpose(0, 2, 4, 1, 3)
        .reshape(b * odim, K * K, (dx // K) * (dy // K))
    )
    r4 = (
        route.reshape(b * idim * odim, dx // K, K, dy // K, K)
        .transpose(0, 2, 4, 1, 3)
        .reshape(b * idim * odim, K * K, (dx // K) * (dy // K))
    )
    perm = jax.random.permutation(jax.random.key(42), N_SEL).astype(jnp.int32)
    perm2d = jnp.broadcast_to(
        jnp.pad(perm, (0, 8), constant_values=-1)[:, None], (32, 64)
    )
    sel = _compute_sel(a4, r4, perm2d)
    votes_r = votes.reshape(b * idim * odim, h, dx * dy)
    out = _gather_votes(votes_r, sel)
    return out[..., :N_SEL].reshape(b, idim, odim, h, N_SEL, 1)


# speedup vs baseline: 3.3970x; 3.3970x over previous
import jax
import jax.numpy as jnp
from jax import lax
from jax.experimental import pallas as pl
from jax.experimental.pallas import tpu as pltpu
from jax.experimental.pallas import tpu_sc as plsc


A_SZ = 16
N_SEL = 24
K = 2

_PERM = tuple(
    int(v) for v in jax.random.permutation(jax.random.key(42), N_SEL)
)
_INVPERM = tuple(_PERM.index(t) for t in range(N_SEL))

_NW = 32
_NPW = 8192 // _NW
_G = 8
_NG = _NPW // _G

_BIG = 3.0e38
_HI = lax.Precision.HIGHEST


def _pool(v, is_base, lff):
    r1 = pltpu.roll(v, 255, 1)
    r16 = pltpu.roll(v, 240, 1)
    r17 = pltpu.roll(v, 239, 1)
    m2 = jnp.maximum(jnp.maximum(v, r1), jnp.maximum(r16, r17))
    mB = jnp.where(is_base, m2, -_BIG)
    wmaxb = jnp.maximum(
        jnp.maximum(mB, pltpu.roll(mB, 1, 1)),
        jnp.maximum(pltpu.roll(mB, 16, 1), pltpu.roll(mB, 17, 1)),
    )
    t = jnp.where(v == wmaxb, jnp.broadcast_to(lff, v.shape), 4096.0)
    tm = jnp.minimum(
        jnp.minimum(t, pltpu.roll(t, 255, 1)),
        jnp.minimum(pltpu.roll(t, 240, 1), pltpu.roll(t, 239, 1)),
    )
    return m2, tm


def _select_body(a_ref, r_ref, selbt_ref, wt_ref, o_ref):
    lanef = lax.broadcasted_iota(jnp.int32, (1, 256), 1)
    is_base = ((lanef % 2) == 0) & ((lanef // 16) % 2 == 0)
    lff = lanef.astype(jnp.float32)
    selbt = selbt_ref[...]

    a = a_ref[...]
    am2, atm = _pool(a, is_base, lff)
    apT = lax.dot_general(selbt, am2, (((1,), (1,)), ((), ())), precision=_HI)
    wio32 = lax.broadcasted_iota(jnp.int32, (64, 32), 0).astype(jnp.float32)
    v = apT
    acc = jnp.zeros((64, 32), jnp.float32)
    for _ in range(A_SZ):
        m = jnp.max(v, axis=0, keepdims=True)
        wsel = jnp.min(jnp.where(v == m, wio32, 64.0), axis=0, keepdims=True)
        oh = wio32 == wsel
        acc = acc + oh.astype(jnp.float32)
        v = jnp.where(oh, -_BIG, v)
    sel16exp = lax.dot_general(
        acc, wt_ref[...], (((0,), (0,)), ((), ())), precision=_HI
    )
    atmB = jnp.where(is_base, atm, 4096.0)
    atmfull = jnp.minimum(
        jnp.minimum(atmB, pltpu.roll(atmB, 1, 1)),
        jnp.minimum(pltpu.roll(atmB, 16, 1), pltpu.roll(atmB, 17, 1)),
    )
    winner = jnp.broadcast_to(lff, (32, 256)) == atmfull
    maskf = sel16exp * winner.astype(jnp.float32)

    r = r_ref[...]
    rm = r + jnp.broadcast_to(maskf[None], (8, 32, 256)).reshape(256, 256)
    rm2, rtm = _pool(rm, is_base, lff)
    rpT = lax.dot_general(selbt, rm2, (((1,), (1,)), ((), ())), precision=_HI)
    rargT = lax.dot_general(selbt, rtm, (((1,), (1,)), ((), ())), precision=_HI)
    wio = lax.broadcasted_iota(jnp.int32, (64, 256), 0).astype(jnp.float32)
    v = rpT
    rows = [None] * 32
    for t_i in range(N_SEL):
        m = jnp.max(v, axis=0, keepdims=True)
        wsel = jnp.min(jnp.where(v == m, wio, 64.0), axis=0, keepdims=True)
        oh = wio == wsel
        rows[_INVPERM[t_i]] = jnp.sum(
            jnp.where(oh, rargT, 0.0), axis=0, keepdims=True
        )
        v = jnp.where(oh, -_BIG, v)
    zero = jnp.zeros((1, 256), jnp.float32)
    for j in range(N_SEL, 32):
        rows[j] = zero
    o_ref[...] = jnp.concatenate(rows, axis=0).astype(jnp.int32)


def _compute_sel(af, rf, selbt, wt, interpret=False):
    return pl.pallas_call(
        _select_body,
        grid=(32,),
        in_specs=[
            pl.BlockSpec((32, 256), lambda t: (t // 4, 0)),
            pl.BlockSpec((256, 256), lambda t: (t, 0)),
            pl.BlockSpec((64, 256), lambda t: (0, 0)),
            pl.BlockSpec((64, 256), lambda t: (0, 0)),
        ],
        out_specs=pl.BlockSpec((32, 256), lambda t: (0, t)),
        out_shape=jax.ShapeDtypeStruct((32, 8192), jnp.int32),
        interpret=interpret,
    )(af, rf, selbt, wt)


def _sc_gather_body(votes_hbm, sel_hbm, out_hbm, idxv, vbuf, obuf, sem_in, sem_out):
    c = lax.axis_index("c")
    s = lax.axis_index("s")
    wid = s * 2 + c
    base = wid * _NPW
    pltpu.sync_copy(sel_hbm.at[:, pl.ds(base, _NPW)], idxv)
    pltpu.async_copy(votes_hbm.at[pl.ds(base, _G)], vbuf.at[0], sem_in)
    jlane = jnp.arange(16, dtype=jnp.int32)
    jlane2 = 16 + (jlane % 8)
    jmask = jlane < 8

    def do_group(g, par):
        @pl.when(g + 1 < _NG)
        def _():
            pltpu.async_copy(
                votes_hbm.at[pl.ds(base + (g + 1) * _G, _G)], vbuf.at[1 - par], sem_in
            )

        pltpu.make_async_copy(
            votes_hbm.at[pl.ds(base + g * _G, _G)], vbuf.at[par], sem_in
        ).wait()

        @pl.when(g >= 2)
        def _():
            pltpu.make_async_copy(
                obuf.at[par], out_hbm.at[pl.ds(base + (g - 2) * _G, _G)], sem_out
            ).wait()

        for e in range(_G):
            n_local = g * _G + e
            nsp = jnp.full((16,), n_local, jnp.int32)
            ir0 = plsc.load_gather(idxv, [jlane, nsp])
            ir1 = plsc.load_gather(idxv, [jlane2, nsp])
            vtab = vbuf.at[par, e]
            for h in range(16):
                hsp = jnp.full((16,), h, jnp.int32)
                obuf[par, e, h, pl.ds(0, 16)] = plsc.load_gather(vtab, [hsp, ir0])
                g1 = plsc.load_gather(vtab, [hsp, ir1])
                plsc.store_scatter(obuf.at[par, e, h], [jlane2], g1, mask=jmask)
        pltpu.async_copy(obuf.at[par], out_hbm.at[pl.ds(base + g * _G, _G)], sem_out)

    def it_body(i, carry):
        do_group(2 * i, 0)
        do_group(2 * i + 1, 1)
        return carry

    lax.fori_loop(0, _NG // 2, it_body, 0)
    pltpu.make_async_copy(
        obuf.at[0], out_hbm.at[pl.ds(base + (_NG - 2) * _G, _G)], sem_out
    ).wait()
    pltpu.make_async_copy(
        obuf.at[1], out_hbm.at[pl.ds(base + (_NG - 1) * _G, _G)], sem_out
    ).wait()


def _gather_votes(votes_r, sel):
    mesh = plsc.VectorSubcoreMesh(core_axis_name="c", subcore_axis_name="s")
    f = pl.kernel(
        _sc_gather_body,
        out_type=jax.ShapeDtypeStruct((8192, 16, N_SEL), jnp.float32),
        mesh=mesh,
        scratch_types=[
            pltpu.VMEM((32, _NPW), jnp.int32),
            pltpu.VMEM((2, _G, 16, 256), jnp.float32),
            pltpu.VMEM((2, _G, 16, N_SEL), jnp.float32),
            pltpu.SemaphoreType.DMA,
            pltpu.SemaphoreType.DMA,
        ],
        compiler_params=pltpu.CompilerParams(
            use_tc_tiling_on_sc=False, needs_layout_passes=False
        ),
    )
    return f(votes_r, sel)


def _constants():
    cc = jnp.arange(256, dtype=jnp.int32)
    ww = jnp.arange(64, dtype=jnp.int32)
    base = 32 * (ww // 8) + 2 * (ww % 8)
    selbt = (base[:, None] == cc[None, :]).astype(jnp.float32)
    win_of_c = (cc // 32) * 8 + (cc % 16) // 2
    wt = (ww[:, None] == win_of_c[None, :]).astype(jnp.float32)
    return selbt, wt


@jax.jit
def kernel(x, route, votes):
    b, idim, odim, h, dx, dy = votes.shape
    af = x[..., h - 1].reshape(b * odim, dx * dy)
    rf = route.reshape(b * idim * odim, dx * dy)
    selbt, wt = _constants()
    sel = _compute_sel(af, rf, selbt, wt)
    votes_r = votes.reshape(b * idim * odim, h, dx * dy)
    out = _gather_votes(votes_r, sel)
    return out.reshape(b, idim, odim, h, N_SEL, 1)
